# baseline (speedup 1.0000x reference)
import jax
import jax.numpy as jnp
from jax.experimental import pallas as pl
from jax.experimental.pallas import tpu as pltpu

N_TOK = 32768
MODEL_DIM = 4096
MAX_POOL = 64
BLK = 512


def _gate_kernel(x_ref, sim_ref, gates_ref, mask_ref, out_ref, topk_ref):
    x = x_ref[...]
    rnorm = jnp.sqrt(jnp.sum(x * x, axis=1, keepdims=True))
    xn = x / jnp.maximum(rnorm, 1e-12)
    s = sim_ref[...]
    cnorm = jnp.sqrt(jnp.sum(s * s, axis=0, keepdims=True))
    sn = s / jnp.maximum(cnorm, 1e-12)
    dots = jnp.dot(xn, sn, preferred_element_type=jnp.float32)
    logits = jax.nn.sigmoid(dots) * mask_ref[...]
    thr = jax.nn.sigmoid(gates_ref[...])
    out = (logits > thr).astype(jnp.float32)
    out_ref[...] = out
    topk_ref[...] = jnp.sum(out, axis=1, keepdims=True).astype(jnp.int32)


def kernel(x, sim_matrix, gates, experts_mask):
    gates2 = gates.reshape(1, MAX_POOL)
    mask2 = experts_mask.reshape(1, MAX_POOL)
    grid = (N_TOK // BLK,)
    logits, topk = pl.pallas_call(
        _gate_kernel,
        grid=grid,
        in_specs=[
            pl.BlockSpec((BLK, MODEL_DIM), lambda i: (i, 0)),
            pl.BlockSpec((MODEL_DIM, MAX_POOL), lambda i: (0, 0)),
            pl.BlockSpec((1, MAX_POOL), lambda i: (0, 0)),
            pl.BlockSpec((1, MAX_POOL), lambda i: (0, 0)),
        ],
        out_specs=[
            pl.BlockSpec((BLK, MAX_POOL), lambda i: (i, 0)),
            pl.BlockSpec((BLK, 1), lambda i: (i, 0)),
        ],
        out_shape=[
            jax.ShapeDtypeStruct((N_TOK, MAX_POOL), jnp.float32),
            jax.ShapeDtypeStruct((N_TOK, 1), jnp.int32),
        ],
        compiler_params=pltpu.CompilerParams(
            dimension_semantics=("parallel",),
        ),
    )(x, sim_matrix, gates2, mask2)
    return (logits, topk.reshape(N_TOK))


# speedup vs baseline: 1.1353x; 1.1353x over previous
import jax
import jax.numpy as jnp
from jax.experimental import pallas as pl
from jax.experimental.pallas import tpu as pltpu

N_TOK = 32768
MODEL_DIM = 4096
MAX_POOL = 64
BLK = 1024


def _prep_kernel(sim_ref, gates_ref, sn_ref, thr_ref):
    s = sim_ref[...]
    cnorm = jnp.sqrt(jnp.sum(s * s, axis=0, keepdims=True))
    sn_ref[...] = (s / jnp.maximum(cnorm, 1e-12)).astype(jnp.bfloat16)
    thr_ref[...] = jax.nn.sigmoid(gates_ref[...])


def _gate_kernel(x_ref, sn_ref, thr_ref, mask_ref, out_ref, topk_ref):
    x = x_ref[...]
    rnorm = jnp.sqrt(jnp.sum(x * x, axis=1, keepdims=True))
    rinv = 1.0 / jnp.maximum(rnorm, 1e-12)
    xn = (x * rinv).astype(jnp.bfloat16)
    dots = jnp.dot(xn, sn_ref[...], preferred_element_type=jnp.float32)
    logits = jax.nn.sigmoid(dots) * mask_ref[...]
    out = (logits > thr_ref[...]).astype(jnp.float32)
    out_ref[...] = out
    topk_ref[...] = jnp.sum(out, axis=1, keepdims=True).astype(jnp.int32)


def kernel(x, sim_matrix, gates, experts_mask):
    gates2 = gates.reshape(1, MAX_POOL)
    mask2 = experts_mask.reshape(1, MAX_POOL)
    sn, thr = pl.pallas_call(
        _prep_kernel,
        out_shape=[
            jax.ShapeDtypeStruct((MODEL_DIM, MAX_POOL), jnp.bfloat16),
            jax.ShapeDtypeStruct((1, MAX_POOL), jnp.float32),
        ],
    )(sim_matrix, gates2)
    grid = (N_TOK // BLK,)
    logits, topk = pl.pallas_call(
        _gate_kernel,
        grid=grid,
        in_specs=[
            pl.BlockSpec((BLK, MODEL_DIM), lambda i: (i, 0)),
            pl.BlockSpec((MODEL_DIM, MAX_POOL), lambda i: (0, 0)),
            pl.BlockSpec((1, MAX_POOL), lambda i: (0, 0)),
            pl.BlockSpec((1, MAX_POOL), lambda i: (0, 0)),
        ],
        out_specs=[
            pl.BlockSpec((BLK, MAX_POOL), lambda i: (i, 0)),
            pl.BlockSpec((BLK, 1), lambda i: (i, 0)),
        ],
        out_shape=[
            jax.ShapeDtypeStruct((N_TOK, MAX_POOL), jnp.float32),
            jax.ShapeDtypeStruct((N_TOK, 1), jnp.int32),
        ],
        compiler_params=pltpu.CompilerParams(
            dimension_semantics=("parallel",),
        ),
    )(x, sn, thr, mask2)
    return (logits, topk.reshape(N_TOK))
